# baseline (speedup 1.0000x reference)
import functools

import jax
import jax.numpy as jnp
from jax import lax
from jax.experimental import pallas as pl
from jax.experimental.pallas import tpu as pltpu
from jax.experimental.pallas import tpu_sc as plsc

_NUM_CORES = 2
_NUM_SUBCORES = 16
_NUM_WORKERS = _NUM_CORES * _NUM_SUBCORES

_LANES = 16
_TROW = 8
_TCOL = 128

_CHUNK = 1024
_NBUF = 2


@functools.lru_cache(maxsize=None)
def _build_transpose(V, D):
    n_banks = D // _TROW
    n_full = V // _TCOL
    tail = V - n_full * _TCOL
    mesh = plsc.VectorSubcoreMesh(core_axis_name="c", subcore_axis_name="s")

    q, r = divmod(n_full, _NUM_WORKERS)

    @functools.partial(
        pl.kernel,
        out_type=jax.ShapeDtypeStruct((V * D,), jnp.float32),
        mesh=mesh,
        scratch_types=[
            [[pltpu.VMEM((_TROW, _TCOL), jnp.float32) for _ in range(n_banks)]
             for _ in range(2)],
            [pltpu.VMEM((_TCOL * D,), jnp.float32) for _ in range(2)],
            [pltpu.VMEM((_TROW, tail), jnp.float32) for _ in range(n_banks)],
            [pltpu.SemaphoreType.DMA for _ in range(2)],
            [pltpu.SemaphoreType.DMA for _ in range(2)],
        ],
        compiler_params=pltpu.CompilerParams(needs_layout_passes=False),
    )
    def transpose_kernel(tt_hbm, lin_hbm, in_bufs, out_bufs, tail_bufs,
                         sem_i, sem_o):
        wid = lax.axis_index("s") * _NUM_CORES + lax.axis_index("c")
        j_lo = wid * q + jnp.minimum(wid, r)
        j_cnt = q + jnp.where(wid < r, 1, 0)
        iota = lax.iota(jnp.int32, _LANES)

        def in_copies(j, s):
            off = j * _TCOL
            return [
                pltpu.make_async_copy(
                    tt_hbm.at[pl.ds(b * _TROW, _TROW), pl.ds(off, _TCOL)],
                    in_bufs[s][b], sem_i[s])
                for b in range(n_banks)
            ]

        def out_copy(j, s):
            return pltpu.make_async_copy(
                out_bufs[s], lin_hbm.at[pl.ds(j * _TCOL * D, _TCOL * D)],
                sem_o[s])

        for c in in_copies(j_lo, 0):
            c.start()

        n_pairs = (j_cnt + 1) // 2

        @pl.loop(0, n_pairs)
        def _pair(g):
            for s in range(2):
                gidx = 2 * g + s
                j = j_lo + gidx

                @pl.when(gidx + 1 < j_cnt)
                def _():
                    for c in in_copies(j + 1, 1 - s):
                        c.start()

                @pl.when(gidx < j_cnt)
                def _():
                    for c in in_copies(j, s):
                        c.wait()

                    @pl.when(gidx >= 2)
                    def _():
                        out_copy(j - 2, s).wait()

                    for b in range(n_banks):
                        for f in range(_TROW):
                            feat = b * _TROW + f
                            for rb in range(_TCOL // _LANES):
                                v = in_bufs[s][b][f, pl.ds(rb * _LANES, _LANES)]
                                plsc.store_scatter(
                                    out_bufs[s],
                                    [(rb * _LANES + iota) * D + feat], v)

                    out_copy(j, s).start()

        out_copy(j_lo, 0).wait()
        out_copy(j_lo + 1, 1).wait()

        if tail:
            @pl.when(wid == _NUM_WORKERS - 1)
            def _():
                for b in range(n_banks):
                    pltpu.sync_copy(
                        tt_hbm.at[pl.ds(b * _TROW, _TROW),
                                  pl.ds(n_full * _TCOL, tail)],
                        tail_bufs[b])
                for b in range(n_banks):
                    for f in range(_TROW):
                        feat = b * _TROW + f
                        for rb in range(tail // _LANES):
                            v = tail_bufs[b][f, pl.ds(rb * _LANES, _LANES)]
                            plsc.store_scatter(
                                out_bufs[0],
                                [(rb * _LANES + iota) * D + feat], v)
                pltpu.sync_copy(
                    out_bufs[0].at[pl.ds(0, tail * D)],
                    lin_hbm.at[pl.ds(n_full * _TCOL * D, tail * D)])

    return transpose_kernel


@functools.lru_cache(maxsize=None)
def _build_gather(B, D):
    assert B % (_NUM_WORKERS * _CHUNK * _NBUF) == 0
    b_per_w = B // _NUM_WORKERS
    n_chunks = b_per_w // _CHUNK
    n_groups = n_chunks // _NBUF
    mesh = plsc.VectorSubcoreMesh(core_axis_name="c", subcore_axis_name="s")

    @functools.partial(
        pl.kernel,
        out_type=jax.ShapeDtypeStruct((B, D), jnp.float32),
        mesh=mesh,
        scratch_types=[
            [pltpu.VMEM((_CHUNK,), jnp.int32) for _ in range(_NBUF)],
            [pltpu.VMEM((_CHUNK, D), jnp.float32) for _ in range(_NBUF)],
            [pltpu.SemaphoreType.DMA for _ in range(3 * _NBUF)],
        ],
        compiler_params=pltpu.CompilerParams(use_tc_tiling_on_sc=False),
    )
    def gather_kernel(idx_hbm, table_hbm, out_hbm, idx_bufs, row_bufs, sems):
        wid = lax.axis_index("s") * _NUM_CORES + lax.axis_index("c")
        base = wid * b_per_w
        sem_i = sems[:_NBUF]
        sem_g = sems[_NBUF:2 * _NBUF]
        sem_o = sems[2 * _NBUF:]

        def idx_copy(b, off):
            return pltpu.make_async_copy(
                idx_hbm.at[pl.ds(off, _CHUNK)], idx_bufs[b], sem_i[b])

        def gather(b):
            return pltpu.make_async_copy(
                table_hbm.at[idx_bufs[b]], row_bufs[b], sem_g[b])

        def store(b, off):
            return pltpu.make_async_copy(
                row_bufs[b], out_hbm.at[pl.ds(off, _CHUNK)], sem_o[b])

        for b in range(_NBUF):
            idx_copy(b, base + b * _CHUNK).start()

        @pl.loop(0, n_groups)
        def _group(g):
            off0 = base + g * _NBUF * _CHUNK
            for b in range(_NBUF):
                off = off0 + b * _CHUNK
                idx_copy(b, off).wait()

                @pl.when(g > 0)
                def _():
                    store(b, off).wait()

                gather(b).start()
            for b in range(_NBUF):
                off = off0 + b * _CHUNK
                gather(b).wait()
                store(b, off).start()

                @pl.when(g + 1 < n_groups)
                def _():
                    idx_copy(b, off + _NBUF * _CHUNK).start()

        for b in range(_NBUF):
            store(b, base + b * _CHUNK).wait()

    return gather_kernel


def kernel(t, table):
    n, m = t.shape
    v, d = table.shape
    lin = _build_transpose(v, d)(table.T)
    table_lin = lin.reshape(v, d)
    out = _build_gather(n * m, d)(t.reshape(n * m), table_lin)
    return out.reshape(n, m, d)


# speedup vs baseline: 3.1575x; 3.1575x over previous
import functools

import jax
import jax.numpy as jnp
from jax import lax
from jax.experimental import pallas as pl
from jax.experimental.pallas import tpu as pltpu
from jax.experimental.pallas import tpu_sc as plsc

_NUM_CORES = 2
_NUM_SUBCORES = 16
_NUM_WORKERS = _NUM_CORES * _NUM_SUBCORES

_LANES = 16
_TROW = 8
_TCOL = 128

_CHUNK = 512


@functools.lru_cache(maxsize=None)
def _build(N, M, D, DP):
    n_banks = D // _TROW
    n_jb = _CHUNK // _TCOL
    blocks = N // _CHUNK
    n_units = M * blocks
    units_pw = n_units // _NUM_WORKERS
    assert units_pw * _NUM_WORKERS == n_units
    o2_words = n_banks * n_jb * _TROW * _TCOL
    seg = n_jb * _TROW * _TCOL
    mesh = plsc.VectorSubcoreMesh(core_axis_name="c", subcore_axis_name="s")

    @functools.partial(
        pl.kernel,
        out_type=jax.ShapeDtypeStruct((M * n_banks * (N // _TCOL) * _TROW
                                       * _TCOL,), jnp.float32),
        mesh=mesh,
        scratch_types=[
            [pltpu.VMEM((_CHUNK,), jnp.int32) for _ in range(2)],
            [pltpu.VMEM((_CHUNK, DP), jnp.float32) for _ in range(2)],
            [pltpu.VMEM((o2_words,), jnp.float32) for _ in range(2)],
            [pltpu.SemaphoreType.DMA for _ in range(2)],
            [pltpu.SemaphoreType.DMA for _ in range(2)],
            [pltpu.SemaphoreType.DMA for _ in range(2)],
        ],
        compiler_params=pltpu.CompilerParams(
            use_tc_tiling_on_sc=False, needs_layout_passes=False),
    )
    def lookup_kernel(tt_hbm, table_hbm, out_hbm, idx_bufs, g_bufs, o_bufs,
                      sem_i, sem_g, sem_o):
        wid = lax.axis_index("s") * _NUM_CORES + lax.axis_index("c")
        u0 = wid * units_pw
        iota = lax.iota(jnp.int32, _LANES)

        def idx_off(u):
            uu = u0 + u
            return (uu // blocks) * N + (uu % blocks) * _CHUNK

        def idx_copy(u, s):
            return pltpu.make_async_copy(
                tt_hbm.at[pl.ds(idx_off(u), _CHUNK)], idx_bufs[s], sem_i[s])

        def gather(s):
            return pltpu.make_async_copy(
                table_hbm.at[idx_bufs[s]], g_bufs[s], sem_g[s])

        def out_copies(u, s):
            uu = u0 + u
            b2 = uu // blocks
            jb0 = (uu % blocks) * n_jb
            return [
                pltpu.make_async_copy(
                    o_bufs[s].at[pl.ds(fi * seg, seg)],
                    out_hbm.at[pl.ds(((b2 * n_banks + fi) * (N // _TCOL)
                                      + jb0) * _TROW * _TCOL, seg)],
                    sem_o[s])
                for fi in range(n_banks)
            ]

        rdiv4 = iota // 4
        cmod4 = iota % 4
        svec = cmod4 * _TCOL + rdiv4
        colsv = [[fi * _TROW + fh * 4 + cmod4 for fh in range(2)]
                 for fi in range(n_banks)]

        def transpose(s):
            @plsc.parallel_loop(0, _TCOL // 4)
            def _rg(k):
                for jl in range(n_jb):
                    rows = rdiv4 + (jl * _TCOL + k * 4)
                    for fi in range(n_banks):
                        for fh in range(2):
                            v = plsc.load_gather(
                                g_bufs[s], [rows, colsv[fi][fh]])
                            plsc.store_scatter(
                                o_bufs[s],
                                [svec + (fi * seg + jl * (_TROW * _TCOL)
                                         + fh * 4 * _TCOL + k * 4)], v)

        idx_copy(0, 0).start()
        idx_copy(1, 1).start()

        @pl.loop(0, units_pw // 2)
        def _pair(g):
            for s in range(2):
                u = 2 * g + s
                idx_copy(u, s).wait()
                gather(s).start()

                @pl.when(u >= 1)
                def _():
                    gather(1 - s).wait()

                    @pl.when(u + 1 < units_pw)
                    def _():
                        idx_copy(u + 1, 1 - s).start()

                    @pl.when(u >= 3)
                    def _():
                        for c in out_copies(u - 3, 1 - s):
                            c.wait()

                    transpose(1 - s)
                    for c in out_copies(u - 1, 1 - s):
                        c.start()

        last = units_pw - 1
        gather(last % 2).wait()
        for c in out_copies(last - 2, last % 2):
            c.wait()
        transpose(last % 2)
        for c in out_copies(last, last % 2):
            c.start()
        for c in out_copies(last - 1, 1 - last % 2):
            c.wait()
        for c in out_copies(last, last % 2):
            c.wait()

    return lookup_kernel


def kernel(t, table):
    n, m = t.shape
    d = table.shape[1]
    lin = _build(n, m, d, d)(t.T.reshape(n * m), table)
    out5 = lin.reshape(m, d // _TROW, n // _TCOL, _TROW, _TCOL)
    return out5.transpose(2, 4, 0, 1, 3).reshape(n, m, d)


# speedup vs baseline: 5.0621x; 1.6032x over previous
import functools

import jax
import jax.numpy as jnp
from jax import lax
from jax.experimental import pallas as pl
from jax.experimental.pallas import tpu as pltpu
from jax.experimental.pallas import tpu_sc as plsc

_NUM_CORES = 2
_NUM_SUBCORES = 16
_NUM_WORKERS = _NUM_CORES * _NUM_SUBCORES

_LANES = 16
_TROW = 8
_TCOL = 128

_CHUNK = 512


@functools.lru_cache(maxsize=None)
def _build_detile(V, D):
    n_banks = D // _TROW
    n_full = V // _TCOL
    tail = V - n_full * _TCOL
    mesh = plsc.VectorSubcoreMesh(core_axis_name="c", subcore_axis_name="s")
    q, r = divmod(n_full, _NUM_WORKERS)

    @functools.partial(
        pl.kernel,
        out_type=jax.ShapeDtypeStruct((V * D,), jnp.float32),
        mesh=mesh,
        scratch_types=[
            [[pltpu.VMEM((_TROW, _TCOL), jnp.float32) for _ in range(n_banks)]
             for _ in range(2)],
            [pltpu.VMEM((_TCOL * D,), jnp.float32) for _ in range(2)],
            [pltpu.VMEM((_TROW, tail), jnp.float32) for _ in range(n_banks)],
            [pltpu.SemaphoreType.DMA for _ in range(2)],
            [pltpu.SemaphoreType.DMA for _ in range(2)],
        ],
        compiler_params=pltpu.CompilerParams(needs_layout_passes=False),
    )
    def detile_kernel(tt_hbm, lin_hbm, in_bufs, out_bufs, tail_bufs,
                      sem_i, sem_o):
        wid = lax.axis_index("s") * _NUM_CORES + lax.axis_index("c")
        j_lo = wid * q + jnp.minimum(wid, r)
        j_cnt = q + jnp.where(wid < r, 1, 0)
        iota = lax.iota(jnp.int32, _LANES)
        rdiv4 = iota // 4
        cmod4 = iota % 4
        svec = cmod4 * D + rdiv4

        def in_copies(j, s):
            off = j * _TCOL
            return [
                pltpu.make_async_copy(
                    tt_hbm.at[pl.ds(b * _TROW, _TROW), pl.ds(off, _TCOL)],
                    in_bufs[s][b], sem_i[s])
                for b in range(n_banks)
            ]

        def out_copy(j, s):
            return pltpu.make_async_copy(
                out_bufs[s], lin_hbm.at[pl.ds(j * _TCOL * D, _TCOL * D)],
                sem_o[s])

        def transpose_block(bufs, dst, ncols):
            @plsc.parallel_loop(0, ncols // 4)
            def _cg(k):
                for b in range(n_banks):
                    for fh in range(2):
                        rows = rdiv4 + fh * 4
                        v = plsc.load_gather(bufs[b], [rows, cmod4 + k * 4])
                        plsc.store_scatter(
                            dst, [svec + (k * 4 * D + b * _TROW + fh * 4)],
                            v)

        for c in in_copies(j_lo, 0):
            c.start()

        n_pairs = (j_cnt + 1) // 2

        @pl.loop(0, n_pairs)
        def _pair(g):
            for s in range(2):
                gidx = 2 * g + s
                j = j_lo + gidx

                @pl.when(gidx + 1 < j_cnt)
                def _():
                    for c in in_copies(j + 1, 1 - s):
                        c.start()

                @pl.when(gidx < j_cnt)
                def _():
                    for c in in_copies(j, s):
                        c.wait()

                    @pl.when(gidx >= 2)
                    def _():
                        out_copy(j - 2, s).wait()

                    transpose_block(in_bufs[s], out_bufs[s], _TCOL)
                    out_copy(j, s).start()

        out_copy(j_lo, 0).wait()
        out_copy(j_lo + 1, 1).wait()

        if tail:
            @pl.when(wid == _NUM_WORKERS - 1)
            def _():
                for b in range(n_banks):
                    pltpu.sync_copy(
                        tt_hbm.at[pl.ds(b * _TROW, _TROW),
                                  pl.ds(n_full * _TCOL, tail)],
                        tail_bufs[b])
                transpose_block(tail_bufs, out_bufs[0], tail)
                pltpu.sync_copy(
                    out_bufs[0].at[pl.ds(0, tail * D)],
                    lin_hbm.at[pl.ds(n_full * _TCOL * D, tail * D)])

    return detile_kernel


@functools.lru_cache(maxsize=None)
def _build(N, M, D, DP):
    n_banks = D // _TROW
    n_jb = _CHUNK // _TCOL
    blocks = N // _CHUNK
    n_units = M * blocks
    units_pw = n_units // _NUM_WORKERS
    assert units_pw * _NUM_WORKERS == n_units
    o2_words = n_banks * n_jb * _TROW * _TCOL
    seg = n_jb * _TROW * _TCOL
    mesh = plsc.VectorSubcoreMesh(core_axis_name="c", subcore_axis_name="s")

    @functools.partial(
        pl.kernel,
        out_type=jax.ShapeDtypeStruct((M * n_banks * (N // _TCOL) * _TROW
                                       * _TCOL,), jnp.float32),
        mesh=mesh,
        scratch_types=[
            [pltpu.VMEM((_CHUNK,), jnp.int32) for _ in range(2)],
            [pltpu.VMEM((_CHUNK, DP), jnp.float32) for _ in range(2)],
            [pltpu.VMEM((o2_words,), jnp.float32) for _ in range(2)],
            [pltpu.SemaphoreType.DMA for _ in range(2)],
            [pltpu.SemaphoreType.DMA for _ in range(2)],
            [pltpu.SemaphoreType.DMA for _ in range(2)],
        ],
        compiler_params=pltpu.CompilerParams(
            use_tc_tiling_on_sc=False, needs_layout_passes=False),
    )
    def lookup_kernel(tt_hbm, table_hbm, out_hbm, idx_bufs, g_bufs, o_bufs,
                      sem_i, sem_g, sem_o):
        wid = lax.axis_index("s") * _NUM_CORES + lax.axis_index("c")
        u0 = wid * units_pw
        iota = lax.iota(jnp.int32, _LANES)

        def idx_off(u):
            uu = u0 + u
            return (uu // blocks) * N + (uu % blocks) * _CHUNK

        def idx_copy(u, s):
            return pltpu.make_async_copy(
                tt_hbm.at[pl.ds(idx_off(u), _CHUNK)], idx_bufs[s], sem_i[s])

        def gather(s):
            return pltpu.make_async_copy(
                table_hbm.at[idx_bufs[s]], g_bufs[s], sem_g[s])

        def out_copies(u, s):
            uu = u0 + u
            b2 = uu // blocks
            jb0 = (uu % blocks) * n_jb
            return [
                pltpu.make_async_copy(
                    o_bufs[s].at[pl.ds(fi * seg, seg)],
                    out_hbm.at[pl.ds(((b2 * n_banks + fi) * (N // _TCOL)
                                      + jb0) * _TROW * _TCOL, seg)],
                    sem_o[s])
                for fi in range(n_banks)
            ]

        rdiv4 = iota // 4
        cmod4 = iota % 4
        svec = cmod4 * _TCOL + rdiv4
        colsv = [[fi * _TROW + fh * 4 + cmod4 for fh in range(2)]
                 for fi in range(n_banks)]

        def transpose(s):
            @plsc.parallel_loop(0, _TCOL // 4)
            def _rg(k):
                for jl in range(n_jb):
                    rows = rdiv4 + (jl * _TCOL + k * 4)
                    for fi in range(n_banks):
                        for fh in range(2):
                            v = plsc.load_gather(
                                g_bufs[s], [rows, colsv[fi][fh]])
                            plsc.store_scatter(
                                o_bufs[s],
                                [svec + (fi * seg + jl * (_TROW * _TCOL)
                                         + fh * 4 * _TCOL + k * 4)], v)

        idx_copy(0, 0).start()
        idx_copy(1, 1).start()

        @pl.loop(0, units_pw // 2)
        def _pair(g):
            for s in range(2):
                u = 2 * g + s
                idx_copy(u, s).wait()
                gather(s).start()

                @pl.when(u >= 1)
                def _():
                    gather(1 - s).wait()

                    @pl.when(u + 1 < units_pw)
                    def _():
                        idx_copy(u + 1, 1 - s).start()

                    @pl.when(u >= 3)
                    def _():
                        for c in out_copies(u - 3, 1 - s):
                            c.wait()

                    transpose(1 - s)
                    for c in out_copies(u - 1, 1 - s):
                        c.start()

        last = units_pw - 1
        gather(last % 2).wait()
        for c in out_copies(last - 2, last % 2):
            c.wait()
        transpose(last % 2)
        for c in out_copies(last, last % 2):
            c.start()
        for c in out_copies(last - 1, 1 - last % 2):
            c.wait()
        for c in out_copies(last, last % 2):
            c.wait()

    return lookup_kernel


def kernel(t, table):
    n, m = t.shape
    v, d = table.shape
    table_lin = _build_detile(v, d)(table.T).reshape(v, d)
    lin = _build(n, m, d, d)(t.T.reshape(n * m), table_lin)
    out5 = lin.reshape(m, d // _TROW, n // _TCOL, _TROW, _TCOL)
    return out5.transpose(2, 4, 0, 1, 3).reshape(n, m, d)
